# Initial kernel scaffold; baseline (speedup 1.0000x reference)
import jax
import jax.numpy as jnp
from jax.experimental import pallas as pl


def kernel(x, edge_index, W1, b1, W2, b2, W3, b3, W4, b4):
    raise NotImplementedError("write your pallas kernel here")


# speedup vs baseline: 9.2494x; 9.2494x over previous
import functools

import jax
import jax.numpy as jnp
from jax import lax
from jax.experimental import pallas as pl
from jax.experimental.pallas import tpu as pltpu
from jax.experimental.pallas import tpu_sc as plsc

N = 50000
E = 800000
NP = 50176
Q = NP // 4
EP = 802816
TILES = 32
NBLK = 196
BLK = 128
EPT = NBLK * BLK
QT = Q // 16
ZR = 98
RD = 4
RBLK = 1024
NC = 6

_MESH = dict(core_axis_name="c", subcore_axis_name="s", num_cores=2,
             num_subcores=16)
_PARAMS = pltpu.CompilerParams(use_tc_tiling_on_sc=False,
                               needs_layout_passes=False)


@functools.cache
def _part_kernel():
    mesh = plsc.VectorSubcoreMesh(**_MESH)
    scratch = [
        pltpu.VMEM((EPT,), jnp.int32),
        pltpu.VMEM((4, EPT + 16), jnp.int32),
        pltpu.VMEM((3, 16), jnp.int32),
        pltpu.SemaphoreType.DMA,
    ]
    out_type = (
        jax.ShapeDtypeStruct((4, TILES, EPT), jnp.int32),
        jax.ShapeDtypeStruct((TILES, 3, 16), jnp.int32),
    )

    def body(pk, dums_h, q_out, cnt_out, inb, qb, cntb, sem):
        cid = lax.axis_index("c")
        sid = lax.axis_index("s")
        wid = sid * 2 + cid
        pltpu.sync_copy(dums_h, qb)
        pltpu.sync_copy(pk.at[wid], inb)

        def step(i, offs):
            o0, o1, o2, o3 = offs
            v = inb[pl.ds(i * 16, 16)]
            d = v & 0xFFFF
            m0 = d < Q
            m1 = jnp.logical_and(d >= Q, d < 2 * Q)
            m2 = jnp.logical_and(d >= 2 * Q, d < 3 * Q)
            m3 = d >= 3 * Q
            plsc.store_compressed(qb.at[0, pl.ds(o0, 16)], v, mask=m0)
            plsc.store_compressed(qb.at[1, pl.ds(o1, 16)], v, mask=m1)
            plsc.store_compressed(qb.at[2, pl.ds(o2, 16)], v, mask=m2)
            plsc.store_compressed(qb.at[3, pl.ds(o3, 16)], v, mask=m3)
            n0 = plsc.all_reduce_population_count(m0)[0]
            n1 = plsc.all_reduce_population_count(m1)[0]
            n2 = plsc.all_reduce_population_count(m2)[0]
            n3 = plsc.all_reduce_population_count(m3)[0]
            return o0 + n0, o1 + n1, o2 + n2, o3 + n3

        c0, c1, c2, _ = lax.fori_loop(0, EPT // 16, step, (0, 0, 0, 0))
        one = jnp.full((16,), 1, jnp.int32)
        cntb[0] = one * c0
        cntb[1] = one * c1
        cntb[2] = one * c2
        for q in range(4):
            pltpu.sync_copy(qb.at[q, pl.ds(0, EPT)], q_out.at[q, wid])
        pltpu.sync_copy(cntb, cnt_out.at[wid])

    return pl.kernel(body, out_type=out_type, mesh=mesh,
                     scratch_types=scratch, compiler_params=_PARAMS)


@functools.cache
def _agg_kernel():
    mesh = plsc.VectorSubcoreMesh(**_MESH)
    scratch = [
        pltpu.VMEM((2, NBLK, BLK), jnp.int32),
        pltpu.VMEM((2, NBLK, BLK), jnp.int32),
        pltpu.VMEM((RD, BLK, 16), jnp.float32),
        pltpu.VMEM((ZR, 16), jnp.float32),
        pltpu.VMEM((TILES, 3, 16), jnp.int32),
        pltpu.VMEM((16,), jnp.int32),
        pltpu.VMEM_SHARED((Q + 16, 16), jnp.float32),
        pltpu.SemaphoreType.DMA,
    ]
    out_type = jax.ShapeDtypeStruct((NC, NP, 16), jnp.float32)

    def body(table, q4, cnt_h, zeros_h, nca_h, out,
             idx_s, idx_d, rows, zbuf, cnts, nca_s, acc, sem):
        cid = lax.axis_index("c")
        sid = lax.axis_index("s")
        pltpu.sync_copy(zeros_h, zbuf)
        pltpu.sync_copy(cnt_h, cnts)
        pltpu.sync_copy(nca_h, nca_s)
        nca = nca_s[...][0]

        for p in range(2):
            base = (2 * cid + p) * Q
            segs = []
            for k in range(2):
                seg = 2 * sid + k
                c0 = cnts[seg, 0, pl.ds(0, 16)][0]
                c1 = cnts[seg, 1, pl.ds(0, 16)][0]
                c2 = cnts[seg, 2, pl.ds(0, 16)][0]
                if p == 0:
                    cnt = jnp.where(cid == 0, c0, c2)
                else:
                    cnt = jnp.where(cid == 0, c1, EPT - c0 - c1 - c2)
                nb = (cnt + BLK - 1) // BLK
                for li in range(2):

                    @pl.when(cid == li)
                    def _stage(li=li, seg=seg, k=k):
                        pltpu.sync_copy(q4.at[2 * li + p, seg], idx_d.at[k])

                def unpack(i, _, k=k):
                    v = idx_d[k, i // 8, pl.ds((i % 8) * 16, 16)]
                    idx_s[k, i // 8, pl.ds((i % 8) * 16, 16)] = (
                        lax.shift_right_logical(v, 16)
                    )
                    idx_d[k, i // 8, pl.ds((i % 8) * 16, 16)] = (
                        (v & 0xFFFF) - base
                    )
                    return 0

                lax.fori_loop(0, nb * 8, unpack, 0)
                segs.append((k, nb))

            def chunk(c, _, segs=segs, base=base):
                tbl = table.at[c]
                for z in range(8):
                    pltpu.sync_copy(zbuf, acc.at[pl.ds(sid * QT + z * ZR, ZR)])
                plsc.subcore_barrier()

                for k, nb in segs:
                    for d in range(RD):

                        @pl.when(d < nb)
                        def _prime(d=d, k=k):
                            pltpu.async_copy(
                                tbl.at[idx_s.at[k, d]], rows.at[d], sem
                            )

                    def sweep(j, _, k=k, nb=nb):
                        slot = lax.rem(j, RD)
                        pltpu.make_async_copy(
                            tbl.at[idx_s.at[0, 0]], rows.at[slot], sem
                        ).wait()
                        pltpu.sync_copy(
                            rows.at[slot], acc.at[idx_d.at[k, j]], add=True
                        )

                        @pl.when(j + RD < nb)
                        def _issue():
                            pltpu.async_copy(
                                tbl.at[idx_s.at[k, j + RD]], rows.at[slot], sem
                            )

                        return 0

                    lax.fori_loop(0, nb, sweep, 0)

                plsc.subcore_barrier()
                pltpu.sync_copy(
                    acc.at[pl.ds(sid * QT, QT)],
                    out.at[c, pl.ds(base + sid * QT, QT)],
                )
                plsc.subcore_barrier()
                return 0

            lax.fori_loop(0, nca, chunk, 0)

    return pl.kernel(body, out_type=out_type, mesh=mesh,
                     scratch_types=scratch, compiler_params=_PARAMS)


@functools.cache
def _deg_kernel():
    mesh = plsc.VectorSubcoreMesh(**_MESH)
    scratch = [
        pltpu.VMEM((EPT,), jnp.int32),
        pltpu.VMEM((NP,), jnp.float32),
        pltpu.SemaphoreType.DMA,
    ]
    out_type = jax.ShapeDtypeStruct((TILES, NP), jnp.float32)

    def body(pk, zeros_n, out, inb, local, sem):
        cid = lax.axis_index("c")
        sid = lax.axis_index("s")
        wid = sid * 2 + cid
        pltpu.sync_copy(zeros_n, local)
        pltpu.sync_copy(pk.at[wid], inb)
        ones16 = jnp.full((16,), 1.0, jnp.float32)

        def step(i, _):
            iv = inb[pl.ds(i * 16, 16)] & 0xFFFF
            plsc.addupdate_scatter(local, [iv], ones16)
            return 0

        lax.fori_loop(0, EPT // 16, step, 0)
        pltpu.sync_copy(local, out.at[wid])

    return pl.kernel(body, out_type=out_type, mesh=mesh,
                     scratch_types=scratch, compiler_params=_PARAMS)


def _dinv_kernel(deg_parts, pad_deg):
    def body(d_ref, p_ref, o_ref):
        o_ref[...] = lax.rsqrt(
            jnp.sum(d_ref[...], axis=0, keepdims=True) - p_ref[...] + 1.0
        )

    return pl.pallas_call(
        body,
        grid=(NP // RBLK,),
        in_specs=[
            pl.BlockSpec((TILES, RBLK), lambda i: (0, i)),
            pl.BlockSpec((1, RBLK), lambda i: (0, i)),
        ],
        out_specs=pl.BlockSpec((1, RBLK), lambda i: (0, i)),
        out_shape=jax.ShapeDtypeStruct((1, NP), jnp.float32),
    )(deg_parts, pad_deg)


def _mm_chunked(h, W, dinv_c):
    def body(x_ref, w_ref, dinv_ref, o_ref):
        t = dinv_ref[...] * jnp.dot(
            x_ref[...], w_ref[...], preferred_element_type=jnp.float32
        )
        for c in range(NC):
            o_ref[c] = t[:, c * 16:(c + 1) * 16]

    return pl.pallas_call(
        body,
        grid=(NP // RBLK,),
        in_specs=[
            pl.BlockSpec((RBLK, NC * 16), lambda i: (i, 0)),
            pl.BlockSpec((NC * 16, NC * 16), lambda i: (0, 0)),
            pl.BlockSpec((RBLK, 1), lambda i: (i, 0)),
        ],
        out_specs=pl.BlockSpec((NC, RBLK, 16), lambda i: (0, i, 0)),
        out_shape=jax.ShapeDtypeStruct((NC, NP, 16), jnp.float32),
    )(h, W, dinv_c)


def _combine(agg, g, dinv_c, bp, cmask, rf):
    def body(agg_ref, g_ref, dinv_ref, b_ref, m_ref, rf_ref, o_ref):
        a = agg_ref[...] + g_ref[...]
        cat = jnp.concatenate([a[c] for c in range(NC)], axis=-1)
        out = dinv_ref[...] * cat + b_ref[...]
        out = jnp.where(rf_ref[...] > 0.0, jnp.maximum(out, 0.0), out)
        o_ref[...] = jnp.where(m_ref[...] > 0.0, out, 0.0)

    return pl.pallas_call(
        body,
        grid=(NP // RBLK,),
        in_specs=[
            pl.BlockSpec((NC, RBLK, 16), lambda i: (0, i, 0)),
            pl.BlockSpec((NC, RBLK, 16), lambda i: (0, i, 0)),
            pl.BlockSpec((RBLK, 1), lambda i: (i, 0)),
            pl.BlockSpec((1, NC * 16), lambda i: (0, 0)),
            pl.BlockSpec((1, NC * 16), lambda i: (0, 0)),
            pl.BlockSpec((1, 1), lambda i: (0, 0)),
        ],
        out_specs=pl.BlockSpec((RBLK, NC * 16), lambda i: (i, 0)),
        out_shape=jax.ShapeDtypeStruct((NP, NC * 16), jnp.float32),
    )(agg, g, dinv_c, bp, cmask, rf)


@jax.jit
def kernel(x, edge_index, W1, b1, W2, b2, W3, b3, W4, b4):
    src = edge_index[0].astype(jnp.int32)
    dst = edge_index[1].astype(jnp.int32)
    sent = jnp.int32(N)
    pk = jnp.concatenate([
        (src << 16) | dst,
        jnp.broadcast_to((sent << 16) | sent, (EP - E,)),
    ]).reshape(TILES, EPT)
    dums_h = jnp.stack([
        jnp.broadcast_to((sent << 16) | jnp.int32((q + 1) * Q), (EPT + 16,))
        for q in range(4)
    ])
    zeros_h = jnp.zeros((ZR, 16), jnp.float32)
    zeros_n = jnp.zeros((NP,), jnp.float32)

    q4, cnts = _part_kernel()(pk, dums_h)
    q4 = q4.reshape(4, TILES, NBLK, BLK)

    deg_parts = _deg_kernel()(pk, zeros_n)
    pad_deg = jnp.zeros((1, NP), jnp.float32).at[0, N].set(float(EP - E))
    dinv_c = _dinv_kernel(deg_parts, pad_deg)[0][:, None]

    CP = NC * 16
    ws, bs, masks = [], [], []
    for W, b in ((W1, b1), (W2, b2), (W3, b3), (W4, b4)):
        fin, fout = W.shape
        ws.append(jnp.pad(W, ((0, CP - fin), (0, CP - fout))))
        bs.append(jnp.pad(b, (0, CP - fout)).reshape(1, CP))
        masks.append((jnp.arange(CP) < fout).astype(jnp.float32).reshape(1, CP))
    W_stack = jnp.stack(ws)
    b_stack = jnp.stack(bs)
    m_stack = jnp.stack(masks)
    rf_stack = jnp.array([1.0, 0.0, 1.0, 0.0], jnp.float32).reshape(4, 1, 1)
    nca_stack = jnp.tile(jnp.array([[5], [4], [5], [6]], jnp.int32), (1, 16))

    h0 = jnp.pad(x, ((0, NP - N), (0, CP - x.shape[1])))

    def step(i, h):
        g = _mm_chunked(h, W_stack[i], dinv_c)
        agg = _agg_kernel()(g, q4, cnts, zeros_h, nca_stack[i])
        return _combine(agg, g, dinv_c, b_stack[i], m_stack[i], rf_stack[i])

    n_layers = 4 + (edge_index[0, 0] >> 31)
    h = lax.fori_loop(0, n_layers, step, h0)
    return h[:N, :88]
